# baseline (speedup 1.0000x reference)
import functools
import math

import jax
import jax.numpy as jnp
from jax.experimental import pallas as pl
from jax.experimental.pallas import tpu as pltpu


def _gcn_kernel(a_ref, in_ref, w1_ref, w2_ref, o_ref, s2_ref, *, bm):
    p = pl.program_id(0)
    i = pl.program_id(1)

    @pl.when(p == 0)
    def _():
        h = jnp.dot(a_ref[...], in_ref[...],
                    preferred_element_type=jnp.float32)
        x = jnp.maximum(jnp.dot(h, w1_ref[...],
                                preferred_element_type=jnp.float32), 0.0)
        s2_ref[pl.ds(pl.multiple_of(i * bm, bm), bm), :] = jnp.dot(
            x, w2_ref[...], preferred_element_type=jnp.float32)

    @pl.when(p == 1)
    def _():
        o_ref[...] = jnp.dot(a_ref[...], s2_ref[...],
                             preferred_element_type=jnp.float32)


def kernel(inputs, adj, W1, W2):
    n, d_in = inputs.shape
    d_hid = W1.shape[1]
    d_out = W2.shape[1]

    BM = 200
    ni = math.ceil(n / BM)

    out = pl.pallas_call(
        functools.partial(_gcn_kernel, bm=BM),
        grid=(2, ni),
        in_specs=[
            pl.BlockSpec((BM, n), lambda p, i: (i, 0)),
            pl.BlockSpec((n, d_in), lambda p, i: (0, 0)),
            pl.BlockSpec((d_in, d_hid), lambda p, i: (0, 0)),
            pl.BlockSpec((d_hid, d_out), lambda p, i: (0, 0)),
        ],
        out_specs=pl.BlockSpec((BM, d_out), lambda p, i: (p * i, 0)),
        out_shape=jax.ShapeDtypeStruct((n, d_out), jnp.float32),
        scratch_shapes=[pltpu.VMEM((n, d_out), jnp.float32)],
        compiler_params=pltpu.CompilerParams(
            dimension_semantics=("arbitrary", "arbitrary")),
    )(adj, inputs, W1, W2)

    return out


# speedup vs baseline: 1.0404x; 1.0404x over previous
import functools
import math

import jax
import jax.numpy as jnp
from jax.experimental import pallas as pl
from jax.experimental.pallas import tpu as pltpu


def _gcn_kernel(a_ref, in_ref, w1_ref, w2_ref, o_ref, s2_ref, *, bm):
    p = pl.program_id(0)
    i = pl.program_id(1)

    @pl.when(p == 0)
    def _():
        h = jnp.dot(a_ref[...], in_ref[...],
                    preferred_element_type=jnp.float32)
        x = jnp.maximum(jnp.dot(h, w1_ref[...],
                                preferred_element_type=jnp.float32), 0.0)
        s2_ref[pl.ds(pl.multiple_of(i * bm, bm), bm), :] = jnp.dot(
            x, w2_ref[...], preferred_element_type=jnp.float32)

    @pl.when(p == 1)
    def _():
        o_ref[...] = jnp.dot(a_ref[...], s2_ref[...],
                             preferred_element_type=jnp.float32)


def kernel(inputs, adj, W1, W2):
    n, d_in = inputs.shape
    d_hid = W1.shape[1]
    d_out = W2.shape[1]

    BM = 400
    ni = math.ceil(n / BM)

    out = pl.pallas_call(
        functools.partial(_gcn_kernel, bm=BM),
        grid=(2, ni),
        in_specs=[
            pl.BlockSpec((BM, n), lambda p, i: (i, 0)),
            pl.BlockSpec((n, d_in), lambda p, i: (0, 0)),
            pl.BlockSpec((d_in, d_hid), lambda p, i: (0, 0)),
            pl.BlockSpec((d_hid, d_out), lambda p, i: (0, 0)),
        ],
        out_specs=pl.BlockSpec((BM, d_out), lambda p, i: (p * i, 0)),
        out_shape=jax.ShapeDtypeStruct((n, d_out), jnp.float32),
        scratch_shapes=[pltpu.VMEM((n, d_out), jnp.float32)],
        compiler_params=pltpu.CompilerParams(
            dimension_semantics=("arbitrary", "arbitrary")),
    )(adj, inputs, W1, W2)

    return out
